# baseline (speedup 1.0000x reference)
import functools

import jax
import jax.numpy as jnp
from jax import lax
from jax.experimental import pallas as pl
from jax.experimental.pallas import tpu as pltpu
from jax.experimental.pallas import tpu_sc as plsc

B = 4096 * 200
NW = 32
B_PER_W = B // NW
CHUNK = 128
N_CHUNKS = B_PER_W // CHUNK
N_ITERS = N_CHUNKS // 4
COLS = (1, 0, 6, 5)


def _body(tab_hbm, xt_hbm, out_hbm,
          ob0, ob1, ob2, ob3,
          xb0, xb1, xb2, xb3,
          ib0, ib1, ib2, ib3,
          g0, g1, g2, g3, w0, w1, w2, w3, xs0, xs1, xs2, xs3):
    obufs = (ob0, ob1, ob2, ob3)
    xbufs = (xb0, xb1, xb2, xb3)
    ibufs = (ib0, ib1, ib2, ib3)
    gsems = (g0, g1, g2, g3)
    wsems = (w0, w1, w2, w3)
    xsems = (xs0, xs1, xs2, xs3)
    wid = lax.axis_index("s") * 2 + lax.axis_index("c")
    base = wid * B_PER_W

    def xcopies(k, s):
        start = base + k * CHUNK
        return [
            pltpu.make_async_copy(
                xt_hbm.at[pl.ds(col * B + start, CHUNK)],
                xbufs[s].at[pl.ds(i * CHUNK, CHUNK)], xsems[s])
            for i, col in enumerate(COLS)
        ]

    def gather(k, s):
        return pltpu.make_async_copy(tab_hbm.at[ibufs[s]], obufs[s], gsems[s])

    def write(k, s):
        return pltpu.make_async_copy(
            obufs[s], out_hbm.at[pl.ds(base + k * CHUNK, CHUNK)], wsems[s])

    def compute_c(s):
        xr, ir = xbufs[s], ibufs[s]
        for g in range(8):
            sl = pl.ds(g * 16, 16)
            a = g * 16
            cc = (xr[pl.ds(a, 16)] + (xr[pl.ds(CHUNK + a, 16)] << 2)
                  + (xr[pl.ds(2 * CHUNK + a, 16)] << 4)
                  + (xr[pl.ds(3 * CHUNK + a, 16)] << 6))
            ir[sl] = cc

    def stage(k, s):
        for cp in xcopies(k, s):
            cp.wait()
        compute_c(s)
        gather(k, s).start()

    for s in range(4):
        for cp in xcopies(s, s):
            cp.start()
    for s in range(2):
        stage(s, s)

    def body4(j, carry):
        for b in range(4):
            k = 4 * j + b
            s = b
            sp = (b + 2) % 4

            if b >= 2:
                write(k - 2, sp).wait()
            else:
                @pl.when(j > 0)
                def _():
                    write(k - 2, sp).wait()

            if b < 2:
                stage(k + 2, sp)
            else:
                @pl.when(j < N_ITERS - 1)
                def _():
                    stage(k + 2, sp)

            @pl.when(j < N_ITERS - 1)
            def _():
                for cp in xcopies(k + 4, b):
                    cp.start()

            gather(k, s).wait()
            write(k, s).start()
        return carry

    lax.fori_loop(0, N_ITERS, body4, 0)
    write(N_CHUNKS - 2, 2).wait()
    write(N_CHUNKS - 1, 3).wait()


@jax.jit
def kernel(x, street_emb, action_emb, position_emb):
    xt = x.reshape(B, 7).astype(jnp.int32).T.reshape(7 * B)

    i = jnp.arange(256, dtype=jnp.int32)
    tab = jnp.concatenate(
        (
            street_emb[i & 3],
            street_emb[(i >> 2) & 3],
            action_emb[(i >> 4) & 3],
            position_emb[(i >> 6) & 3],
        ),
        axis=1,
    )

    mesh = plsc.VectorSubcoreMesh(core_axis_name="c", subcore_axis_name="s")
    run = functools.partial(
        pl.kernel,
        mesh=mesh,
        out_type=jax.ShapeDtypeStruct((B, 128), jnp.float32),
        scratch_types=(
            [pltpu.VMEM((CHUNK, 128), jnp.float32)] * 4
            + [pltpu.VMEM((4 * CHUNK,), jnp.int32)] * 4
            + [pltpu.VMEM((CHUNK,), jnp.int32)] * 4
            + [pltpu.SemaphoreType.DMA] * 12
        ),
    )(_body)
    out = run(tab, xt)
    return out.reshape(4096, 200, 128)


# speedup vs baseline: 1.2966x; 1.2966x over previous
import functools

import jax
import jax.numpy as jnp
from jax import lax
from jax.experimental import pallas as pl
from jax.experimental.pallas import tpu as pltpu
from jax.experimental.pallas import tpu_sc as plsc

B = 4096 * 200
NW = 32
B_PER_W = B // NW
CHUNK = 128
N_CHUNKS = B_PER_W // CHUNK
N_ITERS = N_CHUNKS // 4


def _gather_body(tab_hbm, c_hbm, out_hbm, idx_v,
                 buf0, buf1, buf2, buf3,
                 g0, g1, g2, g3, w0, w1, w2, w3):
    bufs = (buf0, buf1, buf2, buf3)
    gsems = (g0, g1, g2, g3)
    wsems = (w0, w1, w2, w3)
    wid = lax.axis_index("s") * 2 + lax.axis_index("c")
    base = wid * B_PER_W
    pltpu.sync_copy(c_hbm.at[wid], idx_v)

    def gather(k, s):
        return pltpu.make_async_copy(tab_hbm.at[idx_v.at[k]], bufs[s], gsems[s])

    def write(k, s):
        return pltpu.make_async_copy(
            bufs[s], out_hbm.at[pl.ds(base + k * CHUNK, CHUNK)], wsems[s])

    gather(0, 0).start()
    gather(1, 1).start()

    def body4(j, carry):
        for b in range(4):
            k = 4 * j + b
            s = b
            sp = (b + 2) % 4

            if b >= 2:
                write(k - 2, sp).wait()
            else:
                @pl.when(j > 0)
                def _():
                    write(k - 2, sp).wait()

            if b < 2:
                gather(k + 2, sp).start()
            else:
                @pl.when(j < N_ITERS - 1)
                def _():
                    gather(k + 2, sp).start()

            gather(k, s).wait()
            write(k, s).start()
        return carry

    lax.fori_loop(0, N_ITERS, body4, 0)
    write(N_CHUNKS - 2, 2).wait()
    write(N_CHUNKS - 1, 3).wait()


@jax.jit
def kernel(x, street_emb, action_emb, position_emb):
    x32 = x.reshape(B, 7).astype(jnp.int32)
    c = (x32[:, 1] + 4 * x32[:, 0] + 16 * x32[:, 6] + 64 * x32[:, 5])
    c = c.reshape(NW, N_CHUNKS, CHUNK)

    i = jnp.arange(256, dtype=jnp.int32)
    tab = jnp.concatenate(
        (
            street_emb[i & 3],
            street_emb[(i >> 2) & 3],
            action_emb[(i >> 4) & 3],
            position_emb[(i >> 6) & 3],
        ),
        axis=1,
    )

    mesh = plsc.VectorSubcoreMesh(core_axis_name="c", subcore_axis_name="s")
    run = functools.partial(
        pl.kernel,
        mesh=mesh,
        out_type=jax.ShapeDtypeStruct((B, 128), jnp.float32),
        scratch_types=[
            pltpu.VMEM((N_CHUNKS, CHUNK), jnp.int32),
            pltpu.VMEM((CHUNK, 128), jnp.float32),
            pltpu.VMEM((CHUNK, 128), jnp.float32),
            pltpu.VMEM((CHUNK, 128), jnp.float32),
            pltpu.VMEM((CHUNK, 128), jnp.float32),
            pltpu.SemaphoreType.DMA,
            pltpu.SemaphoreType.DMA,
            pltpu.SemaphoreType.DMA,
            pltpu.SemaphoreType.DMA,
            pltpu.SemaphoreType.DMA,
            pltpu.SemaphoreType.DMA,
            pltpu.SemaphoreType.DMA,
            pltpu.SemaphoreType.DMA,
        ],
    )(_gather_body)
    out = run(tab, c)
    return out.reshape(4096, 200, 128)
